# baseline (speedup 1.0000x reference)
import functools

import jax
import jax.numpy as jnp
from jax import lax
from jax.experimental import pallas as pl
from jax.experimental.pallas import tpu as pltpu
from jax.experimental.pallas import tpu_sc as plsc

_N = 10000
_K = 32
_D = 128
_NW = 32
_RPW = 320
_NPAD = _NW * _RPW
_CB = 4
_GB = _CB * _K
_NCH = _RPW // _CB
_NV = _D // 16


def _sc_gather_sum(y, idx2d):
  mesh = plsc.VectorSubcoreMesh(core_axis_name="c", subcore_axis_name="s")

  @functools.partial(
      pl.kernel,
      mesh=mesh,
      out_type=pltpu.HBM((_NPAD, _D), jnp.float32),
      scratch_types=[
          pltpu.VMEM((_NCH, _GB), jnp.int32),
          pltpu.VMEM((2, _GB, _D), jnp.float32),
          pltpu.VMEM((2, 2 * _CB, _D), jnp.float32),
          pltpu.VMEM_SHARED((_N, _D), jnp.float32),
          pltpu.SemaphoreType.DMA,
          pltpu.SemaphoreType.DMA,
          pltpu.SemaphoreType.DMA,
          pltpu.SemaphoreType.DMA,
      ],
  )
  def ksum(idx_hbm, y_hbm, out_hbm, idx_v, rows_v, gout_v, yspm,
           sem0, sem1, semo0, semo1):
    wid = lax.axis_index("s") * 2 + lax.axis_index("c")
    sid = lax.axis_index("s")
    @pl.when(sid < 15)
    def _():
      pltpu.sync_copy(y_hbm.at[pl.ds(sid * 624, 624)],
                      yspm.at[pl.ds(sid * 624, 624)])

    @pl.when(sid == 15)
    def _():
      pltpu.sync_copy(y_hbm.at[pl.ds(9360, 640)],
                      yspm.at[pl.ds(9360, 640)])
    pltpu.sync_copy(idx_hbm.at[pl.ds(wid * _NCH, _NCH)], idx_v)
    plsc.subcore_barrier()

    sems = (sem0, sem1)
    semos = (semo0, semo1)
    _PAIRS = _NCH // 2

    def issue(ch, b):
      pltpu.async_copy(yspm.at[idx_v.at[ch]], rows_v.at[b], sems[b])

    def wait(ch, b):
      pltpu.make_async_copy(yspm.at[idx_v.at[ch]], rows_v.at[b],
                            sems[b]).wait()

    def out_slice(p):
      return out_hbm.at[pl.ds(wid * _RPW + p * 2 * _CB, 2 * _CB)]

    issue(0, 0)
    issue(1, 1)

    def outer(pp, carry):
      for b2 in range(2):
        p = pp * 2 + b2
        @pl.when(pp >= 1)
        def _(p=p, b2=b2):
          pltpu.make_async_copy(gout_v.at[b2], out_slice(p - 2),
                                semos[b2]).wait()
        for b in range(2):
          ch = p * 2 + b
          wait(ch, b)
          for r in range(_CB):
            def red(kk, accs, r=r, b=b):
              out = accs
              for u in range(4):
                j = r * _K + kk * 4 + u
                out = tuple(
                    out[v] + rows_v[b, j, pl.ds(v * 16, 16)]
                    for v in range(_NV)
                )
              return out
            accs = lax.fori_loop(
                0, _K // 4, red,
                tuple(jnp.zeros((16,), jnp.float32) for _ in range(_NV)))
            for v in range(_NV):
              gout_v[b2, b * _CB + r, pl.ds(v * 16, 16)] = accs[v]

          @pl.when(ch + 2 < _NCH)
          def _(ch=ch, b=b):
            issue(ch + 2, b)
        pltpu.async_copy(gout_v.at[b2], out_slice(p), semos[b2])
      return carry

    lax.fori_loop(0, _PAIRS // 2, outer, 0)
    pltpu.make_async_copy(gout_v.at[0], out_slice(_PAIRS - 2), semo0).wait()
    pltpu.make_async_copy(gout_v.at[1], out_slice(_PAIRS - 1), semo1).wait()

  return ksum(idx2d, y)


def _tc_combine(g, y, w, b2d):
  bn = 1000
  grid = (_N // bn,)

  def body(g_ref, y_ref, w_ref, b_ref, o_ref):
    w1 = w_ref[0:_D, :]
    w2 = w_ref[_D:2 * _D, :]
    o_ref[...] = (
        jnp.dot(g_ref[...], w1, preferred_element_type=jnp.float32)
        + jnp.dot(y_ref[...], w2, preferred_element_type=jnp.float32)
        * float(_K)
        + b_ref[...] * float(_K)
    )

  return pl.pallas_call(
      body,
      grid=grid,
      in_specs=[
          pl.BlockSpec((bn, _D), lambda i: (i, 0)),
          pl.BlockSpec((bn, _D), lambda i: (i, 0)),
          pl.BlockSpec((2 * _D, _D), lambda i: (0, 0)),
          pl.BlockSpec((1, _D), lambda i: (0, 0)),
      ],
      out_specs=pl.BlockSpec((bn, _D), lambda i: (i, 0)),
      out_shape=jax.ShapeDtypeStruct((_N, _D), jnp.float32),
  )(g, y, w, b2d)


def kernel(y, neighbor_indices, W, b):
  idx = jnp.where(neighbor_indices == -1, 0, neighbor_indices).astype(
      jnp.int32)
  idx_pad = jnp.zeros((_NPAD, _K), jnp.int32).at[:_N].set(idx)
  idx2d = idx_pad.reshape(_NPAD * _K // _GB, _GB)
  g = _sc_gather_sum(y, idx2d)
  return _tc_combine(g, y, W, b.reshape(1, _D))


# speedup vs baseline: 1.1609x; 1.1609x over previous
import functools

import jax
import jax.numpy as jnp
from jax import lax
from jax.experimental import pallas as pl
from jax.experimental.pallas import tpu as pltpu
from jax.experimental.pallas import tpu_sc as plsc

_N = 10000
_K = 32
_D = 128
_NW = 32
_RPW = 320
_NPAD = _NW * _RPW
_CB = 4
_GB = _CB * _K
_NCH = _RPW // _CB
_NV = _D // 16


def _sc_gather_sum(y, idx2d):
  mesh = plsc.VectorSubcoreMesh(core_axis_name="c", subcore_axis_name="s")

  @functools.partial(
      pl.kernel,
      mesh=mesh,
      compiler_params=pltpu.CompilerParams(needs_layout_passes=False,
                                           use_tc_tiling_on_sc=False),
      out_type=pltpu.HBM((_NPAD, _D), jnp.float32),
      scratch_types=[
          pltpu.VMEM((_NCH, _GB), jnp.int32),
          pltpu.VMEM((2, _GB, _D // 2), jnp.int32),
          pltpu.VMEM((2, 2 * _CB, _D), jnp.float32),
          pltpu.VMEM_SHARED((_N, _D // 2), jnp.int32),
          pltpu.SemaphoreType.DMA,
          pltpu.SemaphoreType.DMA,
          pltpu.SemaphoreType.DMA,
          pltpu.SemaphoreType.DMA,
      ],
  )
  def ksum(idx_hbm, y_hbm, out_hbm, idx_v, rows_v, gout_v, yspm,
           sem0, sem1, semo0, semo1):
    wid = lax.axis_index("s") * 2 + lax.axis_index("c")
    sid = lax.axis_index("s")
    @pl.when(sid < 15)
    def _():
      pltpu.sync_copy(y_hbm.at[pl.ds(sid * 624, 624)],
                      yspm.at[pl.ds(sid * 624, 624)])

    @pl.when(sid == 15)
    def _():
      pltpu.sync_copy(y_hbm.at[pl.ds(9360, 640)],
                      yspm.at[pl.ds(9360, 640)])
    pltpu.sync_copy(idx_hbm.at[pl.ds(wid * _NCH, _NCH)], idx_v)
    plsc.subcore_barrier()

    sems = (sem0, sem1)
    semos = (semo0, semo1)
    _PAIRS = _NCH // 2

    def issue(ch, b):
      pltpu.async_copy(yspm.at[idx_v.at[ch]], rows_v.at[b], sems[b])

    def wait(ch, b):
      pltpu.make_async_copy(yspm.at[idx_v.at[ch]], rows_v.at[b],
                            sems[b]).wait()

    def out_slice(p):
      return out_hbm.at[pl.ds(wid * _RPW + p * 2 * _CB, 2 * _CB)]

    issue(0, 0)
    issue(1, 1)

    def outer(pp, carry):
      for b2 in range(2):
        p = pp * 2 + b2
        @pl.when(pp >= 1)
        def _(p=p, b2=b2):
          pltpu.make_async_copy(gout_v.at[b2], out_slice(p - 2),
                                semos[b2]).wait()
        for b in range(2):
          ch = p * 2 + b
          wait(ch, b)
          for r in range(_CB):
            def red(kk, accs, r=r, b=b):
              out = accs
              for u in range(2):
                j = r * _K + kk * 2 + u
                for m in range(4):
                  raw32 = rows_v[b, j, pl.ds(m * 16, 16)]
                  lo = plsc.bitcast(
                      jnp.left_shift(raw32, jnp.int32(16)), jnp.float32)
                  hi = plsc.bitcast(
                      jnp.bitwise_and(raw32, jnp.int32(-65536)), jnp.float32)
                  out = tuple(
                      (out[v] + lo) if v == 2 * m
                      else ((out[v] + hi) if v == 2 * m + 1 else out[v])
                      for v in range(_NV)
                  )
              return out
            accs = lax.fori_loop(
                0, _K // 2, red,
                tuple(jnp.zeros((16,), jnp.float32) for _ in range(_NV)))
            for v in range(_NV):
              gout_v[b2, b * _CB + r, pl.ds(v * 16, 16)] = accs[v]

          @pl.when(ch + 2 < _NCH)
          def _(ch=ch, b=b):
            issue(ch + 2, b)
        pltpu.async_copy(gout_v.at[b2], out_slice(p), semos[b2])
      return carry

    lax.fori_loop(0, _PAIRS // 2, outer, 0)
    pltpu.make_async_copy(gout_v.at[0], out_slice(_PAIRS - 2), semo0).wait()
    pltpu.make_async_copy(gout_v.at[1], out_slice(_PAIRS - 1), semo1).wait()

  return ksum(idx2d, y)


def _tc_combine(g, y, w, b2d):
  bn = 1000
  grid = (_N // bn,)

  def body(g_ref, y_ref, w_ref, b_ref, o_ref):
    w1 = w_ref[0:_D, :]
    w2 = w_ref[_D:2 * _D, :]
    o_ref[...] = (
        jnp.dot(g_ref[...], w1, preferred_element_type=jnp.float32)
        + jnp.dot(y_ref[...], w2, preferred_element_type=jnp.float32)
        * float(_K)
        + b_ref[...] * float(_K)
    )

  return pl.pallas_call(
      body,
      grid=grid,
      in_specs=[
          pl.BlockSpec((bn, _D), lambda i: (i, 0)),
          pl.BlockSpec((bn, _D), lambda i: (i, 0)),
          pl.BlockSpec((2 * _D, _D), lambda i: (0, 0)),
          pl.BlockSpec((1, _D), lambda i: (0, 0)),
      ],
      out_specs=pl.BlockSpec((bn, _D), lambda i: (i, 0)),
      out_shape=jax.ShapeDtypeStruct((_N, _D), jnp.float32),
  )(g, y, w, b2d)


def kernel(y, neighbor_indices, W, b):
  idx = jnp.where(neighbor_indices == -1, 0, neighbor_indices).astype(
      jnp.int32)
  idx_pad = jnp.zeros((_NPAD, _K), jnp.int32).at[:_N].set(idx)
  idx2d = idx_pad.reshape(_NPAD * _K // _GB, _GB)
  y_perm = y.reshape(_N, 4, 2, 16).transpose(0, 1, 3, 2).reshape(_N, _D)
  ybf = y_perm.astype(jnp.bfloat16)
  ypacked = jax.lax.bitcast_convert_type(
      ybf.reshape(_N, _D // 2, 2), jnp.int32)
  g = _sc_gather_sum(ypacked, idx2d)
  return _tc_combine(g, y, W, b.reshape(1, _D))


# speedup vs baseline: 1.1655x; 1.0040x over previous
import functools

import jax
import jax.numpy as jnp
from jax import lax
from jax.experimental import pallas as pl
from jax.experimental.pallas import tpu as pltpu
from jax.experimental.pallas import tpu_sc as plsc

_N = 10000
_K = 32
_D = 128
_NW = 32
_RPW = 320
_NPAD = _NW * _RPW
_CB = 4
_GB = _CB * _K
_NCH = _RPW // _CB
_NV = _D // 16


def _sc_gather_sum(y, idx2d):
  mesh = plsc.VectorSubcoreMesh(core_axis_name="c", subcore_axis_name="s")

  @functools.partial(
      pl.kernel,
      mesh=mesh,
      compiler_params=pltpu.CompilerParams(needs_layout_passes=False,
                                           use_tc_tiling_on_sc=False),
      out_type=pltpu.HBM((_NPAD, _D), jnp.float32),
      scratch_types=[
          pltpu.VMEM((_NCH, _GB), jnp.int32),
          pltpu.VMEM((4, _GB, _D // 2), jnp.int32),
          pltpu.VMEM((2, 2 * _CB, _D), jnp.float32),
          pltpu.VMEM_SHARED((_N, _D // 2), jnp.int32),
          pltpu.SemaphoreType.DMA,
          pltpu.SemaphoreType.DMA,
          pltpu.SemaphoreType.DMA,
          pltpu.SemaphoreType.DMA,
          pltpu.SemaphoreType.DMA,
          pltpu.SemaphoreType.DMA,
      ],
  )
  def ksum(idx_hbm, y_hbm, out_hbm, idx_v, rows_v, gout_v, yspm,
           sem0, sem1, sem2, sem3, semo0, semo1):
    wid = lax.axis_index("s") * 2 + lax.axis_index("c")
    sid = lax.axis_index("s")
    @pl.when(sid < 15)
    def _():
      pltpu.sync_copy(y_hbm.at[pl.ds(sid * 624, 624)],
                      yspm.at[pl.ds(sid * 624, 624)])

    @pl.when(sid == 15)
    def _():
      pltpu.sync_copy(y_hbm.at[pl.ds(9360, 640)],
                      yspm.at[pl.ds(9360, 640)])
    pltpu.sync_copy(idx_hbm.at[pl.ds(wid * _NCH, _NCH)], idx_v)
    plsc.subcore_barrier()

    sems = (sem0, sem1, sem2, sem3)
    semos = (semo0, semo1)
    _PAIRS = _NCH // 2

    def issue(ch, b):
      pltpu.async_copy(yspm.at[idx_v.at[ch]], rows_v.at[b], sems[b])

    def wait(ch, b):
      pltpu.make_async_copy(yspm.at[idx_v.at[ch]], rows_v.at[b],
                            sems[b]).wait()

    def out_slice(p):
      return out_hbm.at[pl.ds(wid * _RPW + p * 2 * _CB, 2 * _CB)]

    issue(0, 0)
    issue(1, 1)
    issue(2, 2)
    issue(3, 3)

    def outer(pp, carry):
      for b2 in range(2):
        p = pp * 2 + b2
        @pl.when(pp >= 1)
        def _(p=p, b2=b2):
          pltpu.make_async_copy(gout_v.at[b2], out_slice(p - 2),
                                semos[b2]).wait()
        for b in range(2):
          ch = p * 2 + b
          s = 2 * b2 + b
          wait(ch, s)
          for r in range(_CB):
            def red(kk, accs, r=r, s=s):
              out = accs
              for u in range(2):
                j = r * _K + kk * 2 + u
                for m in range(4):
                  raw32 = rows_v[s, j, pl.ds(m * 16, 16)]
                  lo = plsc.bitcast(
                      jnp.left_shift(raw32, jnp.int32(16)), jnp.float32)
                  hi = plsc.bitcast(
                      jnp.bitwise_and(raw32, jnp.int32(-65536)), jnp.float32)
                  out = tuple(
                      (out[v] + lo) if v == 2 * m
                      else ((out[v] + hi) if v == 2 * m + 1 else out[v])
                      for v in range(_NV)
                  )
              return out
            accs = lax.fori_loop(
                0, _K // 2, red,
                tuple(jnp.zeros((16,), jnp.float32) for _ in range(_NV)))
            for v in range(_NV):
              gout_v[b2, b * _CB + r, pl.ds(v * 16, 16)] = accs[v]

          @pl.when(ch + 4 < _NCH)
          def _(ch=ch, s=s):
            issue(ch + 4, s)
        pltpu.async_copy(gout_v.at[b2], out_slice(p), semos[b2])
      return carry

    lax.fori_loop(0, _PAIRS // 2, outer, 0)
    pltpu.make_async_copy(gout_v.at[0], out_slice(_PAIRS - 2), semo0).wait()
    pltpu.make_async_copy(gout_v.at[1], out_slice(_PAIRS - 1), semo1).wait()

  return ksum(idx2d, y)


def _tc_combine(g, y, w, b2d):
  bn = 1000
  grid = (_N // bn,)

  def body(g_ref, y_ref, w_ref, b_ref, o_ref):
    w1 = w_ref[0:_D, :]
    w2 = w_ref[_D:2 * _D, :]
    o_ref[...] = (
        jnp.dot(g_ref[...], w1, preferred_element_type=jnp.float32)
        + jnp.dot(y_ref[...], w2, preferred_element_type=jnp.float32)
        * float(_K)
        + b_ref[...] * float(_K)
    )

  return pl.pallas_call(
      body,
      grid=grid,
      in_specs=[
          pl.BlockSpec((bn, _D), lambda i: (i, 0)),
          pl.BlockSpec((bn, _D), lambda i: (i, 0)),
          pl.BlockSpec((2 * _D, _D), lambda i: (0, 0)),
          pl.BlockSpec((1, _D), lambda i: (0, 0)),
      ],
      out_specs=pl.BlockSpec((bn, _D), lambda i: (i, 0)),
      out_shape=jax.ShapeDtypeStruct((_N, _D), jnp.float32),
  )(g, y, w, b2d)


def kernel(y, neighbor_indices, W, b):
  idx = jnp.where(neighbor_indices == -1, 0, neighbor_indices).astype(
      jnp.int32)
  idx_pad = jnp.zeros((_NPAD, _K), jnp.int32).at[:_N].set(idx)
  idx2d = idx_pad.reshape(_NPAD * _K // _GB, _GB)
  y_perm = y.reshape(_N, 4, 2, 16).transpose(0, 1, 3, 2).reshape(_N, _D)
  ybf = y_perm.astype(jnp.bfloat16)
  ypacked = jax.lax.bitcast_convert_type(
      ybf.reshape(_N, _D // 2, 2), jnp.int32)
  g = _sc_gather_sum(ypacked, idx2d)
  return _tc_combine(g, y, W, b.reshape(1, _D))
